# baseline (speedup 1.0000x reference)
import functools

import jax
import jax.numpy as jnp
from jax import lax
from jax.experimental import pallas as pl
from jax.experimental.pallas import tpu as pltpu
from jax.experimental.pallas import tpu_sc as plsc

NC = 2
NS = 16
NT = NC * NS
CK = 128


def _make_sc_agg(D, CH, SEGP):
    RPS = SEGP // NS

    mesh = plsc.VectorSubcoreMesh(
        core_axis_name="c", subcore_axis_name="s",
        num_cores=NC, num_subcores=NS)

    @functools.partial(
        pl.kernel,
        out_type=[jax.ShapeDtypeStruct((NC * SEGP, D), jnp.float32),
                  jax.ShapeDtypeStruct((NC * SEGP, D), jnp.float32)],
        mesh=mesh,
        scratch_types=[
            pltpu.VMEM((CK,), jnp.int32),
            pltpu.VMEM((CK,), jnp.int32),
            pltpu.VMEM((CK,), jnp.int32),
            pltpu.VMEM((CK, D), jnp.float32),
            pltpu.VMEM((CK, D), jnp.float32),
            pltpu.VMEM_SHARED((SEGP, D), jnp.float32),
            pltpu.VMEM_SHARED((SEGP, D), jnp.float32),
            pltpu.SemaphoreType.DMA,
            pltpu.SemaphoreType.DMA,
            pltpu.SemaphoreType.DMA,
        ],
    )
    def kfn(table, srcI, dstI, z, ones_h, out_agg, out_deg,
            isx, idA, idB, rows, ones_v, agg_sh, deg_sh,
            gsem, asem, dsem):
        c = lax.axis_index("c")
        s = lax.axis_index("s")
        base = (c * NS + s) * CH
        pltpu.sync_copy(ones_h, ones_v)
        pltpu.sync_copy(z, agg_sh.at[pl.ds(s * RPS, RPS)])
        pltpu.sync_copy(z, deg_sh.at[pl.ds(s * RPS, RPS)])
        plsc.subcore_barrier()

        def drain(sem):
            pltpu.make_async_copy(table.at[pl.ds(0, CK)], rows, sem).wait()

        def stage(j, cur_id, nxt_id, first=False, last=False):
            drain(gsem)
            pltpu.async_copy(rows, agg_sh.at[cur_id], asem, add=True)
            pltpu.async_copy(ones_v, deg_sh.at[cur_id], dsem, add=True)
            drain(asem)
            if not first:
                drain(dsem)
            if not last:
                pltpu.sync_copy(srcI.at[base + j + 1], isx)
                pltpu.sync_copy(dstI.at[base + j + 1], nxt_id)
                pltpu.async_copy(table.at[isx], rows, gsem)

        pltpu.sync_copy(srcI.at[base], isx)
        pltpu.sync_copy(dstI.at[base], idA)
        pltpu.async_copy(table.at[isx], rows, gsem)
        stage(0, idA, idB, first=True)

        def body(k, carry):
            stage(2 * k + 1, idB, idA)
            stage(2 * k + 2, idA, idB)
            return carry

        lax.fori_loop(0, (CH - 2) // 2, body, 0)
        stage(CH - 1, idB, idA, last=True)
        drain(dsem)
        plsc.subcore_barrier()
        pltpu.sync_copy(agg_sh.at[pl.ds(s * RPS, RPS)],
                        out_agg.at[pl.ds(c * SEGP + s * RPS, RPS)])
        pltpu.sync_copy(deg_sh.at[pl.ds(s * RPS, RPS)],
                        out_deg.at[pl.ds(c * SEGP + s * RPS, RPS)])

    return kfn


def _dot(a, b):
    return lax.dot_general(a, b, (((1,), (0,)), ((), ())),
                           precision=lax.Precision.HIGHEST,
                           preferred_element_type=jnp.float32)


def _make_tc_linear(NDST, SEGP, D_IN, D_OUT):
    def body(hd_ref, pA_ref, pD_ref, wS_ref, wN_ref, b_ref, out_ref):
        agg = pA_ref[:NDST, :] + pA_ref[SEGP:SEGP + NDST, :]
        deg = pD_ref[:NDST, :1] + pD_ref[SEGP:SEGP + NDST, :1]
        neigh = agg / jnp.maximum(deg, 1.0)
        z = (_dot(hd_ref[:NDST, :], wS_ref[...]) + _dot(neigh, wN_ref[...])
             + b_ref[...])
        out_ref[...] = jnp.maximum(z, 0.0)

    return pl.pallas_call(
        body, out_shape=jax.ShapeDtypeStruct((NDST, D_OUT), jnp.float32))


def _pad_edges(src, dst, segp):
    e = src.shape[0]
    chunks = -(-e // (NT * CK))
    chunks += chunks % 2
    per_tile = chunks * CK
    pad = NT * per_tile - e
    src = jnp.concatenate([src, jnp.zeros((pad,), jnp.int32)])
    dst = jnp.concatenate([dst, jnp.full((pad,), segp - 1, jnp.int32)])
    return src.reshape(NT * chunks, CK), dst.reshape(NT * chunks, CK), chunks


SEG0P = 5120
SEG1P = 1024


def kernel(h, src0, dst0, src1, dst1, num_dst0, num_dst1, W0, b0, W1, b1):
    h = h.astype(jnp.float32)
    src0 = src0.astype(jnp.int32)
    dst0 = dst0.astype(jnp.int32)
    src1 = src1.astype(jnp.int32)
    dst1 = dst1.astype(jnp.int32)

    d_in = h.shape[1]
    d_h = W0.shape[0]
    d_out = W1.shape[0]

    srcI0, dstI0, ch0 = _pad_edges(src0, dst0, SEG0P)
    srcI1, dstI1, ch1 = _pad_edges(src1, dst1, SEG1P)

    z0 = jnp.zeros((SEG0P // NS, d_in), jnp.float32)
    z1 = jnp.zeros((SEG1P // NS, d_h), jnp.float32)
    ones = jnp.ones((CK, d_in), jnp.float32)

    sc0 = _make_sc_agg(d_in, ch0, SEG0P)
    sc1 = _make_sc_agg(d_h, ch1, SEG1P)
    tc0 = _make_tc_linear(5000, SEG0P, d_in, d_h)
    tc1 = _make_tc_linear(1000, SEG1P, d_h, d_out)

    w0S, w0N = W0[:, :d_in].T, W0[:, d_in:].T
    w1S, w1N = W1[:, :d_h].T, W1[:, d_h:].T

    pA0, pD0 = sc0(h, srcI0, dstI0, z0, ones)
    h1 = tc0(h[:5000], pA0, pD0, w0S, w0N, b0.reshape(1, d_h))
    pA1, pD1 = sc1(h1, srcI1, dstI1, z1, ones)
    h2 = tc1(h1[:1000], pA1, pD1, w1S, w1N, b1.reshape(1, d_out))
    return h2


# speedup vs baseline: 1.4143x; 1.4143x over previous
import functools

import jax
import jax.numpy as jnp
from jax import lax
from jax.experimental import pallas as pl
from jax.experimental.pallas import tpu as pltpu
from jax.experimental.pallas import tpu_sc as plsc

NC = 2
NS = 16
NT = NC * NS
CK = 128


def _make_sc_agg(D, CHA, CHB, SEGP):
    RPS = SEGP // NS

    mesh = plsc.VectorSubcoreMesh(
        core_axis_name="c", subcore_axis_name="s",
        num_cores=NC, num_subcores=NS)

    @functools.partial(
        pl.kernel,
        out_type=[jax.ShapeDtypeStruct((NC * SEGP, D), jnp.float32),
                  jax.ShapeDtypeStruct((NC * SEGP, D), jnp.float32)],
        mesh=mesh,
        scratch_types=[
            pltpu.VMEM((CK,), jnp.int32),
            pltpu.VMEM((CK,), jnp.int32),
            pltpu.VMEM((CK, D), jnp.float32),
            pltpu.VMEM((CK, D), jnp.float32),
            pltpu.VMEM_SHARED((SEGP, D), jnp.float32),
            pltpu.VMEM_SHARED((SEGP, D), jnp.float32),
            pltpu.SemaphoreType.DMA,
        ],
    )
    def kfn(table, srcI, dstI, z, ones_h, out_agg, out_deg,
            idx_sc, idx_dc, rows, ones_v, agg_sh, deg_sh, sem):
        c = lax.axis_index("c")
        s = lax.axis_index("s")
        my_ch = jnp.where(c == 0, CHA, CHB)
        base = jnp.where(c == 0, s * CHA, NS * CHA + s * CHB)
        pltpu.sync_copy(ones_h, ones_v)
        pltpu.sync_copy(z, agg_sh.at[pl.ds(s * RPS, RPS)])
        pltpu.sync_copy(z, deg_sh.at[pl.ds(s * RPS, RPS)])
        plsc.subcore_barrier()

        def body(j, carry):
            pltpu.sync_copy(srcI.at[base + j], idx_sc)
            pltpu.sync_copy(dstI.at[base + j], idx_dc)
            pltpu.async_copy(table.at[idx_sc], rows, sem).wait()
            pltpu.sync_copy(rows, agg_sh.at[idx_dc], add=True)
            pltpu.sync_copy(ones_v, deg_sh.at[idx_dc], add=True)
            return carry

        lax.fori_loop(0, my_ch, body, 0)
        plsc.subcore_barrier()
        pltpu.sync_copy(agg_sh.at[pl.ds(s * RPS, RPS)],
                        out_agg.at[pl.ds(c * SEGP + s * RPS, RPS)])
        pltpu.sync_copy(deg_sh.at[pl.ds(s * RPS, RPS)],
                        out_deg.at[pl.ds(c * SEGP + s * RPS, RPS)])

    return kfn


def _dot(a, b):
    return lax.dot_general(a, b, (((1,), (0,)), ((), ())),
                           precision=lax.Precision.HIGHEST,
                           preferred_element_type=jnp.float32)


def _make_tc_linear(NDST, SEGP, D_IN, D_OUT):
    def body(hd_ref, pA_ref, pD_ref, wS_ref, wN_ref, b_ref, out_ref):
        agg = pA_ref[:NDST, :] + pA_ref[SEGP:SEGP + NDST, :]
        deg = pD_ref[:NDST, :1] + pD_ref[SEGP:SEGP + NDST, :1]
        neigh = agg / jnp.maximum(deg, 1.0)
        z = (_dot(hd_ref[:NDST, :], wS_ref[...]) + _dot(neigh, wN_ref[...])
             + b_ref[...])
        out_ref[...] = jnp.maximum(z, 0.0)

    return pl.pallas_call(
        body, out_shape=jax.ShapeDtypeStruct((NDST, D_OUT), jnp.float32))


def _pad_edges(src, dst, segp, frac0):
    e = src.shape[0]
    e0 = (int(e * frac0) // CK) * CK

    def block(s_, d_):
        eb = s_.shape[0]
        ch = -(-eb // (NS * CK))
        pad = NS * ch * CK - eb
        s_ = jnp.concatenate([s_, jnp.zeros((pad,), jnp.int32)])
        d_ = jnp.concatenate([d_, jnp.full((pad,), segp - 1, jnp.int32)])
        return s_.reshape(NS * ch, CK), d_.reshape(NS * ch, CK), ch

    sa, da, cha = block(src[:e0], dst[:e0])
    sb, db, chb = block(src[e0:], dst[e0:])
    return (jnp.concatenate([sa, sb]), jnp.concatenate([da, db]), cha, chb)


SEG0P = 5120
SEG1P = 1024


def kernel(h, src0, dst0, src1, dst1, num_dst0, num_dst1, W0, b0, W1, b1):
    h = h.astype(jnp.float32)
    src0 = src0.astype(jnp.int32)
    dst0 = dst0.astype(jnp.int32)
    src1 = src1.astype(jnp.int32)
    dst1 = dst1.astype(jnp.int32)

    d_in = h.shape[1]
    d_h = W0.shape[0]
    d_out = W1.shape[0]

    srcI0, dstI0, ch0a, ch0b = _pad_edges(src0, dst0, SEG0P, 0.40)
    srcI1, dstI1, ch1a, ch1b = _pad_edges(src1, dst1, SEG1P, 0.35)

    z0 = jnp.zeros((SEG0P // NS, d_in), jnp.float32)
    z1 = jnp.zeros((SEG1P // NS, d_h), jnp.float32)
    ones = jnp.ones((CK, d_in), jnp.float32)

    sc0 = _make_sc_agg(d_in, ch0a, ch0b, SEG0P)
    sc1 = _make_sc_agg(d_h, ch1a, ch1b, SEG1P)
    tc0 = _make_tc_linear(5000, SEG0P, d_in, d_h)
    tc1 = _make_tc_linear(1000, SEG1P, d_h, d_out)

    w0S, w0N = W0[:, :d_in].T, W0[:, d_in:].T
    w1S, w1N = W1[:, :d_h].T, W1[:, d_h:].T

    pA0, pD0 = sc0(h, srcI0, dstI0, z0, ones)
    h1 = tc0(h[:5000], pA0, pD0, w0S, w0N, b0.reshape(1, d_h))
    pA1, pD1 = sc1(h1, srcI1, dstI1, z1, ones)
    h2 = tc1(h1[:1000], pA1, pD1, w1S, w1N, b1.reshape(1, d_out))
    return h2


# speedup vs baseline: 1.4477x; 1.0236x over previous
import functools

import jax
import jax.numpy as jnp
from jax import lax
from jax.experimental import pallas as pl
from jax.experimental.pallas import tpu as pltpu
from jax.experimental.pallas import tpu_sc as plsc

NC = 2
NS = 16
NT = NC * NS
CK = 128


def _make_sc_agg(D, CHA, CHB, SEGP):
    RPS = SEGP // NS

    mesh = plsc.VectorSubcoreMesh(
        core_axis_name="c", subcore_axis_name="s",
        num_cores=NC, num_subcores=NS)

    @functools.partial(
        pl.kernel,
        out_type=[jax.ShapeDtypeStruct((NC * SEGP, D), jnp.float32),
                  jax.ShapeDtypeStruct((NC * SEGP, D), jnp.float32)],
        mesh=mesh,
        scratch_types=[
            pltpu.VMEM((CK,), jnp.int32),
            pltpu.VMEM((CK,), jnp.int32),
            pltpu.VMEM((CK, D), jnp.float32),
            pltpu.VMEM((CK, D), jnp.float32),
            pltpu.VMEM_SHARED((SEGP, D), jnp.float32),
            pltpu.VMEM_SHARED((SEGP, D), jnp.float32),
            pltpu.SemaphoreType.DMA,
        ],
    )
    def kfn(table, srcI, dstI, z, ones_h, out_agg, out_deg,
            idx_sc, idx_dc, rows, ones_v, agg_sh, deg_sh, sem):
        c = lax.axis_index("c")
        s = lax.axis_index("s")
        my_ch = jnp.where(c == 0, CHA, CHB)
        base = jnp.where(c == 0, s * CHA, NS * CHA + s * CHB)
        pltpu.sync_copy(ones_h, ones_v)
        pltpu.sync_copy(z, agg_sh.at[pl.ds(s * RPS, RPS)])
        pltpu.sync_copy(z, deg_sh.at[pl.ds(s * RPS, RPS)])
        plsc.subcore_barrier()

        def body(j, carry):
            pltpu.sync_copy(srcI.at[base + j], idx_sc)
            pltpu.sync_copy(dstI.at[base + j], idx_dc)
            pltpu.async_copy(table.at[idx_sc], rows, sem).wait()
            pltpu.sync_copy(rows, agg_sh.at[idx_dc], add=True)
            pltpu.sync_copy(ones_v, deg_sh.at[idx_dc], add=True)
            return carry

        lax.fori_loop(0, my_ch, body, 0)
        plsc.subcore_barrier()
        pltpu.sync_copy(agg_sh.at[pl.ds(s * RPS, RPS)],
                        out_agg.at[pl.ds(c * SEGP + s * RPS, RPS)])
        pltpu.sync_copy(deg_sh.at[pl.ds(s * RPS, RPS)],
                        out_deg.at[pl.ds(c * SEGP + s * RPS, RPS)])

    return kfn


def _dot(a, b):
    return lax.dot_general(a, b, (((1,), (0,)), ((), ())),
                           precision=lax.Precision.HIGHEST,
                           preferred_element_type=jnp.float32)


def _make_tc_linear(NDST, SEGP, D_IN, D_OUT):
    def body(hd_ref, pA_ref, pD_ref, wS_ref, wN_ref, b_ref, out_ref):
        agg = pA_ref[:NDST, :] + pA_ref[SEGP:SEGP + NDST, :]
        deg = pD_ref[:NDST, :1] + pD_ref[SEGP:SEGP + NDST, :1]
        neigh = agg / jnp.maximum(deg, 1.0)
        z = (_dot(hd_ref[:NDST, :], wS_ref[...]) + _dot(neigh, wN_ref[...])
             + b_ref[...])
        out_ref[...] = jnp.maximum(z, 0.0)

    return pl.pallas_call(
        body, out_shape=jax.ShapeDtypeStruct((NDST, D_OUT), jnp.float32))


def _pad_edges(src, dst, segp, frac0):
    e = src.shape[0]
    e0 = (int(e * frac0) // CK) * CK

    def block(s_, d_):
        eb = s_.shape[0]
        ch = -(-eb // (NS * CK))
        pad = NS * ch * CK - eb
        s_ = jnp.concatenate([s_, jnp.zeros((pad,), jnp.int32)])
        d_ = jnp.concatenate([d_, jnp.full((pad,), segp - 1, jnp.int32)])
        return s_.reshape(NS * ch, CK), d_.reshape(NS * ch, CK), ch

    sa, da, cha = block(src[:e0], dst[:e0])
    sb, db, chb = block(src[e0:], dst[e0:])
    return (jnp.concatenate([sa, sb]), jnp.concatenate([da, db]), cha, chb)


SEG0P = 5120
SEG1P = 1024


def kernel(h, src0, dst0, src1, dst1, num_dst0, num_dst1, W0, b0, W1, b1):
    h = h.astype(jnp.float32)
    src0 = src0.astype(jnp.int32)
    dst0 = dst0.astype(jnp.int32)
    src1 = src1.astype(jnp.int32)
    dst1 = dst1.astype(jnp.int32)

    d_in = h.shape[1]
    d_h = W0.shape[0]
    d_out = W1.shape[0]

    srcI0, dstI0, ch0a, ch0b = _pad_edges(src0, dst0, SEG0P, 0.37)
    srcI1, dstI1, ch1a, ch1b = _pad_edges(src1, dst1, SEG1P, 0.50)

    z0 = jnp.zeros((SEG0P // NS, d_in), jnp.float32)
    z1 = jnp.zeros((SEG1P // NS, d_h), jnp.float32)
    ones = jnp.ones((CK, d_in), jnp.float32)

    sc0 = _make_sc_agg(d_in, ch0a, ch0b, SEG0P)
    sc1 = _make_sc_agg(d_h, ch1a, ch1b, SEG1P)
    tc0 = _make_tc_linear(5000, SEG0P, d_in, d_h)
    tc1 = _make_tc_linear(1000, SEG1P, d_h, d_out)

    w0S, w0N = W0[:, :d_in].T, W0[:, d_in:].T
    w1S, w1N = W1[:, :d_h].T, W1[:, d_h:].T

    pA0, pD0 = sc0(h, srcI0, dstI0, z0, ones)
    h1 = tc0(h[:5000], pA0, pD0, w0S, w0N, b0.reshape(1, d_h))
    pA1, pD1 = sc1(h1, srcI1, dstI1, z1, ones)
    h2 = tc1(h1[:1000], pA1, pD1, w1S, w1N, b1.reshape(1, d_out))
    return h2
